# baseline (speedup 1.0000x reference)
import functools

import jax
import jax.numpy as jnp
from jax.experimental import pallas as pl
from jax.experimental.pallas import tpu as pltpu

_VAR0 = 0.1
_VAR1 = 0.2
_POS_TH = 0.5
_NEG_TH = 0.5


def _body(a_ref, pb_ref, pc_ref, tbt_ref, labr_ref, labc_ref,
          ob_ref, oc_ref, ot_ref, acc_ref, *, nb, nj, n_obj, n_cls):
    b = pl.program_id(0)
    j = pl.program_id(1)

    @pl.when((b == 0) & (j == 0))
    def _init():
        acc_ref[...] = jnp.zeros_like(acc_ref)

    a = a_ref[...]
    acx = a[:, 0:1]
    acy = a[:, 1:2]
    aw = a[:, 2:3]
    ah = a[:, 3:4]
    ax1 = acx - aw / 2.0
    ay1 = acy - ah / 2.0
    ax2 = acx + aw / 2.0
    ay2 = acy + ah / 2.0

    tbt = tbt_ref[0]
    tx1 = tbt[0:1, :]
    ty1 = tbt[1:2, :]
    tx2 = tbt[2:3, :]
    ty2 = tbt[3:4, :]

    ltx = jnp.maximum(ax1, tx1)
    lty = jnp.maximum(ay1, ty1)
    rbx = jnp.minimum(ax2, tx2)
    rby = jnp.minimum(ay2, ty2)
    iw = jnp.clip(rbx - ltx, 0.0, None)
    ih = jnp.clip(rby - lty, 0.0, None)
    inter = iw * ih
    area_a = jnp.clip(ax2 - ax1, 0.0, None) * jnp.clip(ay2 - ay1, 0.0, None)
    area_b = jnp.clip(tx2 - tx1, 0.0, None) * jnp.clip(ty2 - ty1, 0.0, None)
    iou_v = inter / (area_a + area_b - inter + 1e-12)
    labr = labr_ref[0]
    ov = jnp.where(labr < 0, -1.0, iou_v)
    mx = jnp.max(ov, axis=1, keepdims=True)
    pos_b = (jnp.abs(mx - ov) < 1e-6) & (ov > _POS_TH)
    pos = pos_b.astype(jnp.float32)
    neg = (mx < _NEG_TH).astype(jnp.float32)

    pb = pb_ref[0]
    inv_vw = 1.0 / (_VAR0 * aw)
    inv_vh = 1.0 / (_VAR0 * ah)
    g0 = ((tx1 + tx2) / 2.0 - acx) * inv_vw
    g1 = ((ty1 + ty2) / 2.0 - acy) * inv_vh
    lw = jnp.log(tx2 - tx1)
    lh = jnp.log(ty2 - ty1)
    g2 = (lw - jnp.log(aw)) * (1.0 / _VAR1)
    g3 = (lh - jnp.log(ah)) * (1.0 / _VAR1)
    sl1_sum = 0.0
    for i, g in enumerate((g0, g1, g2, g3)):
        d = pb[:, i:i + 1] - g
        ad = jnp.abs(d)
        s = jnp.where(ad < 1.0, 0.5 * d * d, ad - 0.5)
        sl1_sum += jnp.sum(jnp.where(pos_b, s, 0.0))

    pc = pc_ref[0]
    cmx = jnp.max(pc, axis=1, keepdims=True)
    sh = pc - cmx
    lse = jnp.log(jnp.sum(jnp.exp(sh), axis=1, keepdims=True))
    logp = sh - lse
    labc = labc_ref[0]
    oh = (jax.lax.broadcasted_iota(jnp.int32, (n_obj, n_cls), 1)
          == labc).astype(jnp.float32)
    wm = jnp.dot(pos, oh, preferred_element_type=jnp.float32)
    ce_pos = -jnp.sum(wm * logp)
    ce_neg = -jnp.sum(neg * logp[:, 0:1])

    acc_ref[0:1, :] += sl1_sum
    acc_ref[1:2, :] += jnp.sum(pos)
    acc_ref[2:3, :] += ce_pos
    acc_ref[3:4, :] += ce_neg
    acc_ref[4:5, :] += jnp.sum(neg)

    @pl.when((b == nb - 1) & (j == nj - 1))
    def _fin():
        n_pos = acc_ref[1:2, :]
        m = n_pos + acc_ref[4:5, :]
        lb = acc_ref[0:1, :] / jnp.maximum(n_pos, 1.0)
        md = jnp.maximum(m, 1.0)
        lc = (acc_ref[2:3, :] + acc_ref[3:4, :]) / md / md
        ob_ref[...] = lb
        oc_ref[...] = lc
        ot_ref[...] = lb + lc


def kernel(pred_boxes, pred_classes, pred_keypoints, pred_depths,
           tgt_boxes, tgt_keypoints, tgt_depths, anchors, tgt_labels):
    del pred_keypoints, pred_depths, tgt_keypoints, tgt_depths
    nb, n_anch, _ = pred_boxes.shape
    n_obj = tgt_boxes.shape[1]
    n_cls = pred_classes.shape[2]
    blk = 2048
    nj = n_anch // blk

    tbt = jnp.transpose(tgt_boxes, (0, 2, 1))
    labr = jnp.transpose(tgt_labels, (0, 2, 1))

    body = functools.partial(_body, nb=nb, nj=nj, n_obj=n_obj, n_cls=n_cls)
    outs = pl.pallas_call(
        body,
        grid=(nb, nj),
        in_specs=[
            pl.BlockSpec((blk, 4), lambda b, j: (j, 0)),
            pl.BlockSpec((1, blk, 4), lambda b, j: (b, j, 0)),
            pl.BlockSpec((1, blk, n_cls), lambda b, j: (b, j, 0)),
            pl.BlockSpec((1, 4, n_obj), lambda b, j: (b, 0, 0)),
            pl.BlockSpec((1, 1, n_obj), lambda b, j: (b, 0, 0)),
            pl.BlockSpec((1, n_obj, 1), lambda b, j: (b, 0, 0)),
        ],
        out_specs=[
            pl.BlockSpec((1, 128), lambda b, j: (0, 0)),
            pl.BlockSpec((1, 128), lambda b, j: (0, 0)),
            pl.BlockSpec((1, 128), lambda b, j: (0, 0)),
        ],
        out_shape=[jax.ShapeDtypeStruct((1, 128), jnp.float32)] * 3,
        scratch_shapes=[pltpu.VMEM((8, 128), jnp.float32)],
        compiler_params=pltpu.CompilerParams(
            dimension_semantics=("arbitrary", "arbitrary")),
    )(anchors, pred_boxes, pred_classes, tbt, labr, tgt_labels)
    return (outs[0][0, 0], outs[1][0, 0], outs[2][0, 0])


# speedup vs baseline: 2.6436x; 2.6436x over previous
import functools

import jax
import jax.numpy as jnp
from jax.experimental import pallas as pl
from jax.experimental.pallas import tpu as pltpu

_VAR0 = 0.1
_VAR1 = 0.2
_POS_TH = 0.5
_NEG_TH = 0.5


def _body(a_ref, pb_ref, pc_ref, tb_ref, labr_ref, labc_ref,
          ob_ref, oc_ref, ot_ref, acc_ref, *, nb, nj, n_obj, n_cls):
    b = pl.program_id(0)
    j = pl.program_id(1)

    @pl.when((b == 0) & (j == 0))
    def _init():
        acc_ref[...] = jnp.zeros_like(acc_ref)

    acx = a_ref[0:1, :]
    acy = a_ref[1:2, :]
    aw = a_ref[2:3, :]
    ah = a_ref[3:4, :]
    ax1 = acx - aw / 2.0
    ay1 = acy - ah / 2.0
    ax2 = acx + aw / 2.0
    ay2 = acy + ah / 2.0

    tb = tb_ref[0]
    tx1 = tb[:, 0:1]
    ty1 = tb[:, 1:2]
    tx2 = tb[:, 2:3]
    ty2 = tb[:, 3:4]

    ltx = jnp.maximum(ax1, tx1)
    lty = jnp.maximum(ay1, ty1)
    rbx = jnp.minimum(ax2, tx2)
    rby = jnp.minimum(ay2, ty2)
    iw = jnp.clip(rbx - ltx, 0.0, None)
    ih = jnp.clip(rby - lty, 0.0, None)
    inter = iw * ih
    area_a = jnp.clip(ax2 - ax1, 0.0, None) * jnp.clip(ay2 - ay1, 0.0, None)
    area_b = jnp.clip(tx2 - tx1, 0.0, None) * jnp.clip(ty2 - ty1, 0.0, None)
    iou_v = inter / (area_a + area_b - inter + 1e-12)
    labc = labc_ref[0]
    ov = jnp.where(labc < 0, -1.0, iou_v)
    mx = jnp.max(ov, axis=0, keepdims=True)
    pos_b = (jnp.abs(mx - ov) < 1e-6) & (ov > _POS_TH)
    pos = pos_b.astype(jnp.float32)
    neg = (mx < _NEG_TH).astype(jnp.float32)

    pb = pb_ref[0]
    inv_vw = 1.0 / (_VAR0 * aw)
    inv_vh = 1.0 / (_VAR0 * ah)
    g0 = ((tx1 + tx2) / 2.0 - acx) * inv_vw
    g1 = ((ty1 + ty2) / 2.0 - acy) * inv_vh
    lw = jnp.log(tx2 - tx1)
    lh = jnp.log(ty2 - ty1)
    g2 = (lw - jnp.log(aw)) * (1.0 / _VAR1)
    g3 = (lh - jnp.log(ah)) * (1.0 / _VAR1)
    st = None
    for i, g in enumerate((g0, g1, g2, g3)):
        d = pb[i:i + 1, :] - g
        ad = jnp.abs(d)
        s = jnp.where(ad < 1.0, 0.5 * d * d, ad - 0.5)
        st = s if st is None else st + s
    sl1_sum = jnp.sum(jnp.where(pos_b, st, 0.0))

    pc = pc_ref[0]
    bm = jnp.max(pc)
    e = jnp.exp(pc - bm)
    ones_c = jnp.ones((n_cls, 1), jnp.float32)
    lse = jnp.log(jnp.dot(e, ones_c, preferred_element_type=jnp.float32)) + bm
    pc0 = pc[:, 0:1]
    labr = labr_ref[0]
    ohT = (jax.lax.broadcasted_iota(jnp.int32, (n_cls, n_obj), 0)
           == labr).astype(jnp.float32)
    pcg = jnp.dot(pc, ohT, preferred_element_type=jnp.float32)
    pp = jnp.dot(pos, pcg, preferred_element_type=jnp.float32)
    eye = (jax.lax.broadcasted_iota(jnp.int32, (n_obj, n_obj), 0)
           == jax.lax.broadcasted_iota(jnp.int32, (n_obj, n_obj), 1))
    tr = jnp.sum(jnp.where(eye, pp, 0.0))
    cnt = jnp.sum(pos, axis=0, keepdims=True)
    s1 = jnp.dot(cnt, lse, preferred_element_type=jnp.float32)[0, 0]
    ce_pos = s1 - tr
    ce_neg = -jnp.dot(neg, pc0 - lse,
                      preferred_element_type=jnp.float32)[0, 0]

    acc_ref[0:1, :] += sl1_sum
    acc_ref[1:2, :] += jnp.sum(pos)
    acc_ref[2:3, :] += ce_pos
    acc_ref[3:4, :] += ce_neg
    acc_ref[4:5, :] += jnp.sum(neg)

    @pl.when((b == nb - 1) & (j == nj - 1))
    def _fin():
        n_pos = acc_ref[1:2, :]
        m = n_pos + acc_ref[4:5, :]
        lb = acc_ref[0:1, :] / jnp.maximum(n_pos, 1.0)
        md = jnp.maximum(m, 1.0)
        lc = (acc_ref[2:3, :] + acc_ref[3:4, :]) / md / md
        ob_ref[...] = lb
        oc_ref[...] = lc
        ot_ref[...] = lb + lc


def kernel(pred_boxes, pred_classes, pred_keypoints, pred_depths,
           tgt_boxes, tgt_keypoints, tgt_depths, anchors, tgt_labels):
    del pred_keypoints, pred_depths, tgt_keypoints, tgt_depths
    nb, n_anch, _ = pred_boxes.shape
    n_obj = tgt_boxes.shape[1]
    n_cls = pred_classes.shape[2]
    blk = 2048
    nj = n_anch // blk

    a_t = anchors.T
    pb_t = jnp.transpose(pred_boxes, (0, 2, 1))
    labr = jnp.transpose(tgt_labels, (0, 2, 1))

    body = functools.partial(_body, nb=nb, nj=nj, n_obj=n_obj, n_cls=n_cls)
    outs = pl.pallas_call(
        body,
        grid=(nb, nj),
        in_specs=[
            pl.BlockSpec((4, blk), lambda b, j: (0, j)),
            pl.BlockSpec((1, 4, blk), lambda b, j: (b, 0, j)),
            pl.BlockSpec((1, blk, n_cls), lambda b, j: (b, j, 0)),
            pl.BlockSpec((1, n_obj, 4), lambda b, j: (b, 0, 0)),
            pl.BlockSpec((1, 1, n_obj), lambda b, j: (b, 0, 0)),
            pl.BlockSpec((1, n_obj, 1), lambda b, j: (b, 0, 0)),
        ],
        out_specs=[
            pl.BlockSpec((1, 128), lambda b, j: (0, 0)),
            pl.BlockSpec((1, 128), lambda b, j: (0, 0)),
            pl.BlockSpec((1, 128), lambda b, j: (0, 0)),
        ],
        out_shape=[jax.ShapeDtypeStruct((1, 128), jnp.float32)] * 3,
        scratch_shapes=[pltpu.VMEM((8, 128), jnp.float32)],
        compiler_params=pltpu.CompilerParams(
            dimension_semantics=("arbitrary", "arbitrary")),
    )(a_t, pb_t, pred_classes, tgt_boxes, labr, tgt_labels)
    return (outs[0][0, 0], outs[1][0, 0], outs[2][0, 0])


# speedup vs baseline: 3.0111x; 1.1390x over previous
import jax
import jax.numpy as jnp
from jax.experimental import pallas as pl
from jax.experimental.pallas import tpu as pltpu

_VAR0 = 0.1
_VAR1 = 0.2
_POS_TH = 0.5
_NEG_TH = 0.5


def _body(a_ref, pb_ref, pc_ref, tb_ref, labr_ref, labc_ref,
          ob_ref, oc_ref, ot_ref, acc_ref, *, nb, nj, n_obj, n_cls):
    b = pl.program_id(0)
    j = pl.program_id(1)

    @pl.when((b == 0) & (j == 0))
    def _init():
        acc_ref[...] = jnp.zeros_like(acc_ref)

    acx = a_ref[0:1, :]
    acy = a_ref[1:2, :]
    aw = a_ref[2:3, :]
    ah = a_ref[3:4, :]
    ax1 = acx - aw / 2.0
    ay1 = acy - ah / 2.0
    ax2 = acx + aw / 2.0
    ay2 = acy + ah / 2.0

    tb = tb_ref[0]
    tx1 = tb[:, 0:1]
    ty1 = tb[:, 1:2]
    tx2 = tb[:, 2:3]
    ty2 = tb[:, 3:4]

    ltx = jnp.maximum(ax1, tx1)
    lty = jnp.maximum(ay1, ty1)
    rbx = jnp.minimum(ax2, tx2)
    rby = jnp.minimum(ay2, ty2)
    iw = jnp.clip(rbx - ltx, 0.0, None)
    ih = jnp.clip(rby - lty, 0.0, None)
    inter = iw * ih
    area_a = jnp.clip(ax2 - ax1, 0.0, None) * jnp.clip(ay2 - ay1, 0.0, None)
    area_b = jnp.clip(tx2 - tx1, 0.0, None) * jnp.clip(ty2 - ty1, 0.0, None)
    iou_v = inter / (area_a + area_b - inter + 1e-12)
    labc = labc_ref[0]
    ov = jnp.where(labc < 0, -1.0, iou_v)
    mx = jnp.max(ov, axis=0, keepdims=True)
    pos_b = (jnp.abs(mx - ov) < 1e-6) & (ov > _POS_TH)
    pos = pos_b.astype(jnp.float32)
    neg = (mx < _NEG_TH).astype(jnp.float32)

    pb = pb_ref[0]
    inv_vw = 1.0 / (_VAR0 * aw)
    inv_vh = 1.0 / (_VAR0 * ah)
    g0 = ((tx1 + tx2) / 2.0 - acx) * inv_vw
    g1 = ((ty1 + ty2) / 2.0 - acy) * inv_vh
    lw = jnp.log(tx2 - tx1)
    lh = jnp.log(ty2 - ty1)
    g2 = (lw - jnp.log(aw)) * (1.0 / _VAR1)
    g3 = (lh - jnp.log(ah)) * (1.0 / _VAR1)
    sl1_sum = 0.0
    for i, g in enumerate((g0, g1, g2, g3)):
        d = pb[i:i + 1, :] - g
        ad = jnp.abs(d)
        s = jnp.where(ad < 1.0, 0.5 * d * d, ad - 0.5)
        sl1_sum += jnp.sum(jnp.where(pos_b, s, 0.0))

    pc = pc_ref[0]
    cmx = jnp.max(pc, axis=0, keepdims=True)
    sh = pc - cmx
    lse = jnp.log(jnp.sum(jnp.exp(sh), axis=0, keepdims=True))
    logp = sh - lse
    labr = labr_ref[0]
    oh = (jax.lax.broadcasted_iota(jnp.int32, (n_cls, n_obj), 0)
          == labr).astype(jnp.float32)
    wm = jnp.dot(oh, pos, preferred_element_type=jnp.float32)
    ce_pos = -jnp.sum(wm * logp)
    ce_neg = -jnp.sum(neg * logp[0:1, :])

    acc_ref[0:1, :] += sl1_sum
    acc_ref[1:2, :] += jnp.sum(pos)
    acc_ref[2:3, :] += ce_pos
    acc_ref[3:4, :] += ce_neg
    acc_ref[4:5, :] += jnp.sum(neg)

    @pl.when((b == nb - 1) & (j == nj - 1))
    def _fin():
        n_pos = acc_ref[1:2, :]
        m = n_pos + acc_ref[4:5, :]
        lb = acc_ref[0:1, :] / jnp.maximum(n_pos, 1.0)
        md = jnp.maximum(m, 1.0)
        lc = (acc_ref[2:3, :] + acc_ref[3:4, :]) / md / md
        ob_ref[...] = lb
        oc_ref[...] = lc
        ot_ref[...] = lb + lc


def kernel(pred_boxes, pred_classes, pred_keypoints, pred_depths,
           tgt_boxes, tgt_keypoints, tgt_depths, anchors, tgt_labels):
    del pred_keypoints, pred_depths, tgt_keypoints, tgt_depths
    nb, n_anch, _ = pred_boxes.shape
    n_obj = tgt_boxes.shape[1]
    n_cls = pred_classes.shape[2]
    blk = 2048
    nj = n_anch // blk

    a_t = anchors.T
    pb_t = jnp.transpose(pred_boxes, (0, 2, 1))
    pc_t = jnp.transpose(pred_classes, (0, 2, 1))
    labr = jnp.transpose(tgt_labels, (0, 2, 1))

    import functools
    body = functools.partial(_body, nb=nb, nj=nj, n_obj=n_obj, n_cls=n_cls)
    outs = pl.pallas_call(
        body,
        grid=(nb, nj),
        in_specs=[
            pl.BlockSpec((4, blk), lambda b, j: (0, j)),
            pl.BlockSpec((1, 4, blk), lambda b, j: (b, 0, j)),
            pl.BlockSpec((1, n_cls, blk), lambda b, j: (b, 0, j)),
            pl.BlockSpec((1, n_obj, 4), lambda b, j: (b, 0, 0)),
            pl.BlockSpec((1, 1, n_obj), lambda b, j: (b, 0, 0)),
            pl.BlockSpec((1, n_obj, 1), lambda b, j: (b, 0, 0)),
        ],
        out_specs=[
            pl.BlockSpec((1, 128), lambda b, j: (0, 0)),
            pl.BlockSpec((1, 128), lambda b, j: (0, 0)),
            pl.BlockSpec((1, 128), lambda b, j: (0, 0)),
        ],
        out_shape=[jax.ShapeDtypeStruct((1, 128), jnp.float32)] * 3,
        scratch_shapes=[pltpu.VMEM((8, 128), jnp.float32)],
        compiler_params=pltpu.CompilerParams(
            dimension_semantics=("arbitrary", "arbitrary")),
    )(a_t, pb_t, pc_t, tgt_boxes, labr, tgt_labels)
    return (outs[0][0, 0], outs[1][0, 0], outs[2][0, 0])
